# baseline (speedup 1.0000x reference)
import functools

import jax
import jax.numpy as jnp
from jax import lax
from jax.experimental import pallas as pl
from jax.experimental.pallas import tpu as pltpu
from jax.experimental.pallas import tpu_sc as plsc

BM = 256
BH = 1024
_SQRT_HALF = 0.7071067811865476


def _gelu_exact(z):
    return 0.5 * z * (1.0 + lax.erf(z * _SQRT_HALF))


def _router_kernel(x_ref, lat_ref, wr_ref, wl_ref, tri_ref,
                   pos_ref, g1_ref, g2_ref,
                   ve_ref, vb_ref, vh_ref, loss_ref):
    x = x_ref[...]
    logits = jnp.dot(x, wr_ref[...].T, preferred_element_type=jnp.float32)
    logits = logits + jnp.dot(lat_ref[...], wl_ref[...].T,
                              preferred_element_type=jnp.float32)

    m = jnp.max(logits, axis=-1, keepdims=True)
    ex = jnp.exp(logits - m)
    probs = ex / jnp.sum(ex, axis=-1, keepdims=True)

    N, E = probs.shape
    col = lax.broadcasted_iota(jnp.int32, (N, E), 1)

    m1 = jnp.max(probs, axis=-1, keepdims=True)
    a1 = jnp.min(jnp.where(probs == m1, col, E), axis=-1, keepdims=True)
    oh1 = (col == a1).astype(jnp.float32)
    masked = jnp.where(col == a1, -jnp.inf, probs)
    m2 = jnp.max(masked, axis=-1, keepdims=True)
    a2 = jnp.min(jnp.where(masked == m2, col, E), axis=-1, keepdims=True)
    oh2 = (col == a2).astype(jnp.float32)

    denom = m1 + m2
    g1_ref[...] = m1 / denom
    g2_ref[...] = m2 / denom

    tri = tri_ref[...]
    c1 = jnp.dot(tri, oh1.astype(jnp.bfloat16),
                 preferred_element_type=jnp.float32)
    c2 = jnp.dot(tri, oh2.astype(jnp.bfloat16),
                 preferred_element_type=jnp.float32)
    count1 = c1[N - 1:N, :]
    count2 = c2[N - 1:N, :]
    counts = count1 + count2

    padded = jnp.ceil(counts * (1.0 / BM)) * BM
    r8 = lax.broadcasted_iota(jnp.int32, (E, E), 0)
    c8 = lax.broadcasted_iota(jnp.int32, (E, E), 1)
    t8 = (r8 < c8).astype(jnp.float32)
    off_pad = jnp.dot(padded, t8, preferred_element_type=jnp.float32)

    vmax = vb_ref.shape[0]
    ntiles = padded * (1.0 / BM)
    cum_excl = off_pad * (1.0 / BM)
    cum_incl = cum_excl + ntiles
    total = jnp.sum(ntiles, axis=-1, keepdims=True)
    vv = lax.broadcasted_iota(jnp.int32, (vmax, 1), 0).astype(jnp.float32)
    vb = jnp.minimum(vv, total - 1.0)
    ve = jnp.sum((cum_incl <= vb).astype(jnp.float32), axis=-1, keepdims=True)
    colv = lax.broadcasted_iota(jnp.int32, (vmax, E), 1).astype(jnp.float32)
    ohv = (colv == ve).astype(jnp.float32)
    off_v = jnp.sum(off_pad * ohv, axis=-1, keepdims=True)
    cnt_v = jnp.sum(counts * ohv, axis=-1, keepdims=True)
    hi = jnp.minimum(off_v + cnt_v, (vb + 1.0) * BM)
    vh = jnp.where(vv < total, hi, 0.0)
    ve_ref[...] = ve.astype(jnp.int32)
    vb_ref[...] = vb.astype(jnp.int32)
    vh_ref[...] = vh.astype(jnp.int32)

    rank1 = jnp.sum(c1 * oh1, axis=-1, keepdims=True) - 1.0
    pos1 = jnp.sum(off_pad * oh1, axis=-1, keepdims=True) + rank1
    rank2 = jnp.sum((count1 + c2) * oh2, axis=-1, keepdims=True) - 1.0
    pos2 = jnp.sum(off_pad * oh2, axis=-1, keepdims=True) + rank2
    pos_ref[...] = jnp.concatenate([pos1, pos2], axis=0).astype(jnp.int32)

    prob_mean = jnp.mean(probs, axis=0)
    frac_mean = counts[0] * (1.0 / N)
    loss = jnp.float32(E) * jnp.sum(prob_mean * frac_mean)
    loss_ref[...] = loss.reshape(1, 1)


def _route(x_flat, latent_code, Wr, Wl, vmax):
    N, D = x_flat.shape
    E = Wr.shape[0]
    tri = jnp.tril(jnp.ones((N, N), dtype=jnp.bfloat16))
    pos, g1, g2, ve, vb, vh, loss = pl.pallas_call(
        _router_kernel,
        out_shape=(
            jax.ShapeDtypeStruct((2 * N, 1), jnp.int32),
            jax.ShapeDtypeStruct((N, 1), jnp.float32),
            jax.ShapeDtypeStruct((N, 1), jnp.float32),
            jax.ShapeDtypeStruct((vmax, 1), jnp.int32),
            jax.ShapeDtypeStruct((vmax, 1), jnp.int32),
            jax.ShapeDtypeStruct((vmax, 1), jnp.int32),
            jax.ShapeDtypeStruct((1, 1), jnp.float32),
        ),
    )(x_flat, latent_code, Wr, Wl, tri)
    return pos, g1, g2, ve, vb, vh, loss[0, 0]


def _sc_dispatch(pos, x_flat, npad):
    N, D = x_flat.shape
    S = 2 * N
    info = plsc.get_sparse_core_info()
    nc, ns = info.num_cores, info.num_subcores
    nw = nc * ns
    per_w = S // nw
    ch = min(per_w, 64)
    mesh = plsc.VectorSubcoreMesh(core_axis_name="c", subcore_axis_name="s")

    @functools.partial(
        pl.kernel,
        out_type=jax.ShapeDtypeStruct((npad, D), jnp.float32),
        mesh=mesh,
        scratch_types=[
            pltpu.VMEM((ch,), jnp.int32),
            pltpu.VMEM((ch, D), jnp.float32),
            pltpu.SemaphoreType.DMA,
        ],
    )
    def scatter(pos_hbm, x_hbm, xs_hbm, idx_v, rows_v, sem):
        wid = lax.axis_index("s") * nc + lax.axis_index("c")
        base = wid * per_w
        for c in range(per_w // ch):
            b = base + c * ch
            srow = lax.rem(b, N)
            pltpu.sync_copy(pos_hbm.at[pl.ds(b, ch)], idx_v)
            pltpu.sync_copy(x_hbm.at[pl.ds(srow, ch)], rows_v)
            pltpu.async_copy(rows_v, xs_hbm.at[idx_v], sem).wait()

    return scatter(pos, x_flat)


def _sc_collect(pos, ys):
    S = pos.shape[0]
    npad, D = ys.shape
    info = plsc.get_sparse_core_info()
    nc, ns = info.num_cores, info.num_subcores
    nw = nc * ns
    per_w = S // nw
    ch = min(per_w, 64)
    mesh = plsc.VectorSubcoreMesh(core_axis_name="c", subcore_axis_name="s")

    @functools.partial(
        pl.kernel,
        out_type=jax.ShapeDtypeStruct((S, D), jnp.float32),
        mesh=mesh,
        scratch_types=[
            pltpu.VMEM((ch,), jnp.int32),
            pltpu.VMEM((ch, D), jnp.float32),
            pltpu.SemaphoreType.DMA,
        ],
    )
    def gather(pos_hbm, ys_hbm, yp_hbm, idx_v, rows_v, sem):
        wid = lax.axis_index("s") * nc + lax.axis_index("c")
        base = wid * per_w
        for c in range(per_w // ch):
            b = base + c * ch
            pltpu.sync_copy(pos_hbm.at[pl.ds(b, ch)], idx_v)
            pltpu.async_copy(ys_hbm.at[idx_v], rows_v, sem).wait()
            pltpu.sync_copy(rows_v, yp_hbm.at[pl.ds(b, ch)])

    return gather(pos, ys)


def _ffn_kernel(ve_ref, vb_ref, vh_ref,
                xs_ref, w1_ref, b1_ref, w2_ref, b2_ref, ys_ref):
    h = pl.program_id(0)
    v = pl.program_id(1)
    block = vb_ref[v]
    hi = vh_ref[v]

    @pl.when(hi > block * BM)
    def _():
        x = xs_ref[...].astype(jnp.bfloat16)
        w1 = w1_ref[0]
        hidden = jnp.dot(x, w1.T, preferred_element_type=jnp.float32)
        hidden = _gelu_exact(hidden + b1_ref[0, 0]).astype(jnp.bfloat16)
        y = jnp.dot(hidden, w2_ref[0].T, preferred_element_type=jnp.float32)

        rows = block * BM + lax.broadcasted_iota(jnp.int32, (BM, 1), 0)
        mask = rows < hi

        @pl.when(h == 0)
        def _():
            ys_ref[pl.ds(block * BM, BM), :] = jnp.where(mask, y + b2_ref[0], 0.0)

        @pl.when(h != 0)
        def _():
            ys_ref[pl.ds(block * BM, BM), :] += jnp.where(mask, y, 0.0)


def _grouped_ffn(xs, ve, vb, vh, W1, b1, W2, b2):
    npad, D = xs.shape
    E, H, _ = W1.shape
    nh = H // BH
    vmax = ve.shape[0]
    b1r = b1.reshape(E, nh, 1, BH)
    b2r = b2.reshape(E, 1, D)
    grid_spec = pltpu.PrefetchScalarGridSpec(
        num_scalar_prefetch=3,
        grid=(nh, vmax),
        in_specs=[
            pl.BlockSpec((BM, D), lambda h, v, ve, vb, vh: (vb[v], 0)),
            pl.BlockSpec((1, BH, D), lambda h, v, ve, vb, vh: (ve[v], h, 0)),
            pl.BlockSpec((1, 1, 1, BH), lambda h, v, ve, vb, vh: (ve[v], h, 0, 0)),
            pl.BlockSpec((1, D, BH), lambda h, v, ve, vb, vh: (ve[v], 0, h)),
            pl.BlockSpec((1, 1, D), lambda h, v, ve, vb, vh: (ve[v], 0, 0)),
        ],
        out_specs=pl.BlockSpec((npad, D), lambda h, v, ve, vb, vh: (0, 0)),
    )
    return pl.pallas_call(
        _ffn_kernel,
        grid_spec=grid_spec,
        out_shape=jax.ShapeDtypeStruct((npad, D), jnp.float32),
    )(ve, vb, vh, xs, W1, b1r, W2, b2r)


def _combine_kernel(y1_ref, y2_ref, g1_ref, g2_ref, out_ref):
    out_ref[...] = g1_ref[...] * y1_ref[...] + g2_ref[...] * y2_ref[...]


def _combine(yp, g1, g2, bt=512):
    S, D = yp.shape
    N = S // 2
    return pl.pallas_call(
        _combine_kernel,
        grid=(N // bt,),
        in_specs=[
            pl.BlockSpec((bt, D), lambda t: (t, 0)),
            pl.BlockSpec((bt, D), lambda t, n=N // bt: (t + n, 0)),
            pl.BlockSpec((bt, 1), lambda t: (t, 0)),
            pl.BlockSpec((bt, 1), lambda t: (t, 0)),
        ],
        out_specs=pl.BlockSpec((bt, D), lambda t: (t, 0)),
        out_shape=jax.ShapeDtypeStruct((N, D), jnp.float32),
    )(yp, yp, g1, g2)


@jax.jit
def kernel(x, latent_code, Wr, Wl, W1, b1, W2, b2):
    B, T, D = x.shape
    E, H, _ = W1.shape
    x_flat = x.reshape(-1, D)
    N = x_flat.shape[0]
    npad = 2 * N + E * BM
    vmax = npad // BM

    pos, g1, g2, ve, vb, vh, loss = _route(x_flat, latent_code, Wr, Wl, vmax)

    pos_flat = pos.reshape(2 * N)
    xs = _sc_dispatch(pos_flat, x_flat, npad)
    ys = _grouped_ffn(xs, ve.reshape(vmax), vb.reshape(vmax), vh.reshape(vmax),
                      W1.astype(jnp.bfloat16), b1, W2.astype(jnp.bfloat16), b2)
    yp = _sc_collect(pos_flat, ys)
    out = _combine(yp, g1, g2)
    return out.reshape(B, T, D), loss


# speedup vs baseline: 1.3052x; 1.3052x over previous
import functools

import jax
import jax.numpy as jnp
from jax import lax
from jax.experimental import pallas as pl
from jax.experimental.pallas import tpu as pltpu
from jax.experimental.pallas import tpu_sc as plsc

BM = 256
BH = 1024
_SQRT_HALF = 0.7071067811865476


def _gelu_exact(z):
    return 0.5 * z * (1.0 + lax.erf(z * _SQRT_HALF))


def _router_kernel(x_ref, lat_ref, wr_ref, wl_ref, tri_ref,
                   pos_ref, g1_ref, g2_ref,
                   ve_ref, vb_ref, vh_ref, loss_ref):
    x = x_ref[...]
    logits = jnp.dot(x, wr_ref[...].T, preferred_element_type=jnp.float32)
    logits = logits + jnp.dot(lat_ref[...], wl_ref[...].T,
                              preferred_element_type=jnp.float32)

    m = jnp.max(logits, axis=-1, keepdims=True)
    ex = jnp.exp(logits - m)
    probs = ex / jnp.sum(ex, axis=-1, keepdims=True)

    N, E = probs.shape
    col = lax.broadcasted_iota(jnp.int32, (N, E), 1)

    m1 = jnp.max(probs, axis=-1, keepdims=True)
    a1 = jnp.min(jnp.where(probs == m1, col, E), axis=-1, keepdims=True)
    oh1 = (col == a1).astype(jnp.float32)
    masked = jnp.where(col == a1, -jnp.inf, probs)
    m2 = jnp.max(masked, axis=-1, keepdims=True)
    a2 = jnp.min(jnp.where(masked == m2, col, E), axis=-1, keepdims=True)
    oh2 = (col == a2).astype(jnp.float32)

    denom = m1 + m2
    g1_ref[...] = m1 / denom
    g2_ref[...] = m2 / denom

    tri = tri_ref[...]
    c1 = jnp.dot(tri, oh1.astype(jnp.bfloat16),
                 preferred_element_type=jnp.float32)
    c2 = jnp.dot(tri, oh2.astype(jnp.bfloat16),
                 preferred_element_type=jnp.float32)
    count1 = c1[N - 1:N, :]
    count2 = c2[N - 1:N, :]
    counts = count1 + count2

    padded = jnp.ceil(counts * (1.0 / BM)) * BM
    r8 = lax.broadcasted_iota(jnp.int32, (E, E), 0)
    c8 = lax.broadcasted_iota(jnp.int32, (E, E), 1)
    t8 = (r8 < c8).astype(jnp.float32)
    off_pad = jnp.dot(padded, t8, preferred_element_type=jnp.float32)

    vmax = vb_ref.shape[0]
    ntiles = padded * (1.0 / BM)
    cum_excl = off_pad * (1.0 / BM)
    cum_incl = cum_excl + ntiles
    total = jnp.sum(ntiles, axis=-1, keepdims=True)
    vv = lax.broadcasted_iota(jnp.int32, (vmax, 1), 0).astype(jnp.float32)
    vb = jnp.minimum(vv, total - 1.0)
    ve = jnp.sum((cum_incl <= vb).astype(jnp.float32), axis=-1, keepdims=True)
    colv = lax.broadcasted_iota(jnp.int32, (vmax, E), 1).astype(jnp.float32)
    ohv = (colv == ve).astype(jnp.float32)
    off_v = jnp.sum(off_pad * ohv, axis=-1, keepdims=True)
    cnt_v = jnp.sum(counts * ohv, axis=-1, keepdims=True)
    hi = jnp.minimum(off_v + cnt_v, (vb + 1.0) * BM)
    vh = jnp.where(vv < total, hi, 0.0)
    ve_ref[...] = ve.astype(jnp.int32)
    vb_ref[...] = vb.astype(jnp.int32)
    vh_ref[...] = vh.astype(jnp.int32)

    rank1 = jnp.sum(c1 * oh1, axis=-1, keepdims=True) - 1.0
    pos1 = jnp.sum(off_pad * oh1, axis=-1, keepdims=True) + rank1
    rank2 = jnp.sum((count1 + c2) * oh2, axis=-1, keepdims=True) - 1.0
    pos2 = jnp.sum(off_pad * oh2, axis=-1, keepdims=True) + rank2
    pos_ref[...] = jnp.concatenate([pos1, pos2], axis=0).astype(jnp.int32)

    prob_mean = jnp.mean(probs, axis=0)
    frac_mean = counts[0] * (1.0 / N)
    loss = jnp.float32(E) * jnp.sum(prob_mean * frac_mean)
    loss_ref[...] = loss.reshape(1, 1)


def _route(x_flat, latent_code, Wr, Wl, vmax):
    N, D = x_flat.shape
    E = Wr.shape[0]
    tri = jnp.tril(jnp.ones((N, N), dtype=jnp.bfloat16))
    pos, g1, g2, ve, vb, vh, loss = pl.pallas_call(
        _router_kernel,
        out_shape=(
            jax.ShapeDtypeStruct((2 * N, 1), jnp.int32),
            jax.ShapeDtypeStruct((N, 1), jnp.float32),
            jax.ShapeDtypeStruct((N, 1), jnp.float32),
            jax.ShapeDtypeStruct((vmax, 1), jnp.int32),
            jax.ShapeDtypeStruct((vmax, 1), jnp.int32),
            jax.ShapeDtypeStruct((vmax, 1), jnp.int32),
            jax.ShapeDtypeStruct((1, 1), jnp.float32),
        ),
    )(x_flat, latent_code, Wr, Wl, tri)
    return pos, g1, g2, ve, vb, vh, loss[0, 0]


def _sc_dispatch(pos, x_flat, npad):
    N, D = x_flat.shape
    S = 2 * N
    info = plsc.get_sparse_core_info()
    nc, ns = info.num_cores, info.num_subcores
    nw = nc * ns
    per_w = S // nw
    ch = min(per_w, 64)
    mesh = plsc.VectorSubcoreMesh(core_axis_name="c", subcore_axis_name="s")

    @functools.partial(
        pl.kernel,
        out_type=jax.ShapeDtypeStruct((npad, D), jnp.float32),
        mesh=mesh,
        scratch_types=[
            pltpu.VMEM((ch,), jnp.int32),
            pltpu.VMEM((ch, D), jnp.float32),
            pltpu.SemaphoreType.DMA,
        ],
    )
    def scatter(pos_hbm, x_hbm, xs_hbm, idx_v, rows_v, sem):
        wid = lax.axis_index("s") * nc + lax.axis_index("c")
        base = wid * per_w
        for c in range(per_w // ch):
            b = base + c * ch
            srow = lax.rem(b, N)
            pltpu.sync_copy(pos_hbm.at[pl.ds(b, ch)], idx_v)
            pltpu.sync_copy(x_hbm.at[pl.ds(srow, ch)], rows_v)
            pltpu.async_copy(rows_v, xs_hbm.at[idx_v], sem).wait()

    return scatter(pos, x_flat)


def _sc_collect(pos, ys):
    S = pos.shape[0]
    npad, D = ys.shape
    info = plsc.get_sparse_core_info()
    nc, ns = info.num_cores, info.num_subcores
    nw = nc * ns
    per_w = S // nw
    ch = min(per_w, 64)
    mesh = plsc.VectorSubcoreMesh(core_axis_name="c", subcore_axis_name="s")

    @functools.partial(
        pl.kernel,
        out_type=jax.ShapeDtypeStruct((S, D), jnp.float32),
        mesh=mesh,
        scratch_types=[
            pltpu.VMEM((ch,), jnp.int32),
            pltpu.VMEM((ch, D), jnp.float32),
            pltpu.SemaphoreType.DMA,
        ],
    )
    def gather(pos_hbm, ys_hbm, yp_hbm, idx_v, rows_v, sem):
        wid = lax.axis_index("s") * nc + lax.axis_index("c")
        base = wid * per_w
        for c in range(per_w // ch):
            b = base + c * ch
            pltpu.sync_copy(pos_hbm.at[pl.ds(b, ch)], idx_v)
            pltpu.async_copy(ys_hbm.at[idx_v], rows_v, sem).wait()
            pltpu.sync_copy(rows_v, yp_hbm.at[pl.ds(b, ch)])

    return gather(pos, ys)


def _ffn_kernel(ve_ref, vb_ref, vh_ref,
                xs_ref, w1_ref, b1_ref, w2_ref, b2_ref, ys_ref):
    h = pl.program_id(0)
    v = pl.program_id(1)
    block = vb_ref[v]
    hi = vh_ref[v]

    @pl.when(hi > block * BM)
    def _():
        x = xs_ref[...]
        w1 = w1_ref[0]
        hidden = jnp.dot(x, w1.T, preferred_element_type=jnp.float32)
        hidden = _gelu_exact(hidden + b1_ref[0, 0])
        y = jnp.dot(hidden, w2_ref[0].T, preferred_element_type=jnp.float32)

        rows = block * BM + lax.broadcasted_iota(jnp.int32, (BM, 1), 0)
        mask = rows < hi

        @pl.when(h == 0)
        def _():
            ys_ref[pl.ds(block * BM, BM), :] = jnp.where(mask, y + b2_ref[0], 0.0)

        @pl.when(h != 0)
        def _():
            ys_ref[pl.ds(block * BM, BM), :] += jnp.where(mask, y, 0.0)


def _grouped_ffn(xs, ve, vb, vh, W1, b1, W2, b2):
    npad, D = xs.shape
    E, H, _ = W1.shape
    nh = H // BH
    vmax = ve.shape[0]
    b1r = b1.reshape(E, nh, 1, BH)
    b2r = b2.reshape(E, 1, D)
    grid_spec = pltpu.PrefetchScalarGridSpec(
        num_scalar_prefetch=3,
        grid=(nh, vmax),
        in_specs=[
            pl.BlockSpec((BM, D), lambda h, v, ve, vb, vh: (vb[v], 0)),
            pl.BlockSpec((1, BH, D), lambda h, v, ve, vb, vh: (ve[v], h, 0)),
            pl.BlockSpec((1, 1, 1, BH), lambda h, v, ve, vb, vh: (ve[v], h, 0, 0)),
            pl.BlockSpec((1, D, BH), lambda h, v, ve, vb, vh: (ve[v], 0, h)),
            pl.BlockSpec((1, 1, D), lambda h, v, ve, vb, vh: (ve[v], 0, 0)),
        ],
        out_specs=pl.BlockSpec((npad, D), lambda h, v, ve, vb, vh: (0, 0)),
    )
    return pl.pallas_call(
        _ffn_kernel,
        grid_spec=grid_spec,
        out_shape=jax.ShapeDtypeStruct((npad, D), jnp.float32),
    )(ve, vb, vh, xs, W1, b1r, W2, b2r)


def _combine_kernel(y1_ref, y2_ref, g1_ref, g2_ref, out_ref):
    out_ref[...] = g1_ref[...] * y1_ref[...] + g2_ref[...] * y2_ref[...]


def _combine(yp, g1, g2, bt=512):
    S, D = yp.shape
    N = S // 2
    return pl.pallas_call(
        _combine_kernel,
        grid=(N // bt,),
        in_specs=[
            pl.BlockSpec((bt, D), lambda t: (t, 0)),
            pl.BlockSpec((bt, D), lambda t, n=N // bt: (t + n, 0)),
            pl.BlockSpec((bt, 1), lambda t: (t, 0)),
            pl.BlockSpec((bt, 1), lambda t: (t, 0)),
        ],
        out_specs=pl.BlockSpec((bt, D), lambda t: (t, 0)),
        out_shape=jax.ShapeDtypeStruct((N, D), jnp.float32),
    )(yp, yp, g1, g2)


@jax.jit
def kernel(x, latent_code, Wr, Wl, W1, b1, W2, b2):
    B, T, D = x.shape
    E, H, _ = W1.shape
    x_flat = x.reshape(-1, D)
    N = x_flat.shape[0]
    npad = 2 * N + E * BM
    vmax = npad // BM

    pos, g1, g2, ve, vb, vh, loss = _route(x_flat, latent_code, Wr, Wl, vmax)

    pos_flat = pos.reshape(2 * N)
    xs = _sc_dispatch(pos_flat, x_flat, npad)
    ys = _grouped_ffn(xs, ve.reshape(vmax), vb.reshape(vmax), vh.reshape(vmax),
                      W1, b1, W2, b2)
    yp = _sc_collect(pos_flat, ys)
    out = _combine(yp, g1, g2)
    return out.reshape(B, T, D), loss


# speedup vs baseline: 1.5064x; 1.1542x over previous
import functools

import jax
import jax.numpy as jnp
from jax import lax
from jax.experimental import pallas as pl
from jax.experimental.pallas import tpu as pltpu
from jax.experimental.pallas import tpu_sc as plsc

BM = 512
BH = 1024
_SQRT_HALF = 0.7071067811865476


def _gelu_exact(z):
    return 0.5 * z * (1.0 + lax.erf(z * _SQRT_HALF))


def _router_kernel(x_ref, lat_ref, wr_ref, wl_ref, tri_ref,
                   pos_ref, g1_ref, g2_ref,
                   ve_ref, vb_ref, vh_ref, loss_ref):
    x = x_ref[...]
    logits = jnp.dot(x, wr_ref[...].T, preferred_element_type=jnp.float32)
    logits = logits + jnp.dot(lat_ref[...], wl_ref[...].T,
                              preferred_element_type=jnp.float32)

    m = jnp.max(logits, axis=-1, keepdims=True)
    ex = jnp.exp(logits - m)
    probs = ex / jnp.sum(ex, axis=-1, keepdims=True)

    N, E = probs.shape
    col = lax.broadcasted_iota(jnp.int32, (N, E), 1)

    m1 = jnp.max(probs, axis=-1, keepdims=True)
    a1 = jnp.min(jnp.where(probs == m1, col, E), axis=-1, keepdims=True)
    oh1 = (col == a1).astype(jnp.float32)
    masked = jnp.where(col == a1, -jnp.inf, probs)
    m2 = jnp.max(masked, axis=-1, keepdims=True)
    a2 = jnp.min(jnp.where(masked == m2, col, E), axis=-1, keepdims=True)
    oh2 = (col == a2).astype(jnp.float32)

    denom = m1 + m2
    g1_ref[...] = m1 / denom
    g2_ref[...] = m2 / denom

    tri = tri_ref[...]
    c1 = jnp.dot(tri, oh1.astype(jnp.bfloat16),
                 preferred_element_type=jnp.float32)
    c2 = jnp.dot(tri, oh2.astype(jnp.bfloat16),
                 preferred_element_type=jnp.float32)
    count1 = c1[N - 1:N, :]
    count2 = c2[N - 1:N, :]
    counts = count1 + count2

    padded = jnp.ceil(counts * (1.0 / BM)) * BM
    r8 = lax.broadcasted_iota(jnp.int32, (E, E), 0)
    c8 = lax.broadcasted_iota(jnp.int32, (E, E), 1)
    t8 = (r8 < c8).astype(jnp.float32)
    off_pad = jnp.dot(padded, t8, preferred_element_type=jnp.float32)

    vmax = 2 * N // BM + E
    ntiles = padded * (1.0 / BM)
    cum_excl = off_pad * (1.0 / BM)
    cum_incl = cum_excl + ntiles
    total = jnp.sum(ntiles, axis=-1, keepdims=True)
    iota_v = lax.broadcasted_iota(jnp.int32, (vmax, 1), 0).astype(jnp.float32)
    colv = lax.broadcasted_iota(jnp.int32, (vmax, E), 1).astype(jnp.float32)

    def meta(vv):
        vb = jnp.minimum(vv, total - 1.0)
        ve = jnp.sum((cum_incl <= vb).astype(jnp.float32),
                     axis=-1, keepdims=True)
        ohv = (colv == ve).astype(jnp.float32)
        off_v = jnp.sum(off_pad * ohv, axis=-1, keepdims=True)
        cnt_v = jnp.sum(counts * ohv, axis=-1, keepdims=True)
        hi = jnp.minimum(off_v + cnt_v, (vb + 1.0) * BM)
        vh = jnp.where(vv < total, hi, 0.0)
        return (ve.astype(jnp.int32), vb.astype(jnp.int32),
                vh.astype(jnp.int32))

    nh = ve_ref.shape[0] // vmax
    vei, vbi, vhi = meta(iota_v)
    ver, vbr, vhr = meta((vmax - 1.0) - iota_v)
    ve_ref[...] = jnp.concatenate(
        [ver if h % 2 else vei for h in range(nh)], axis=0)
    vb_ref[...] = jnp.concatenate(
        [vbr if h % 2 else vbi for h in range(nh)], axis=0)
    vh_ref[...] = jnp.concatenate(
        [vhr if h % 2 else vhi for h in range(nh)], axis=0)

    rank1 = jnp.sum(c1 * oh1, axis=-1, keepdims=True) - 1.0
    pos1 = jnp.sum(off_pad * oh1, axis=-1, keepdims=True) + rank1
    rank2 = jnp.sum((count1 + c2) * oh2, axis=-1, keepdims=True) - 1.0
    pos2 = jnp.sum(off_pad * oh2, axis=-1, keepdims=True) + rank2
    pos_ref[...] = jnp.concatenate([pos1, pos2], axis=0).astype(jnp.int32)

    prob_mean = jnp.mean(probs, axis=0)
    frac_mean = counts[0] * (1.0 / N)
    loss = jnp.float32(E) * jnp.sum(prob_mean * frac_mean)
    loss_ref[...] = loss.reshape(1, 1)


def _route(x_flat, latent_code, Wr, Wl, vmax, nh):
    N, D = x_flat.shape
    E = Wr.shape[0]
    tri = jnp.tril(jnp.ones((N, N), dtype=jnp.bfloat16))
    pos, g1, g2, ve, vb, vh, loss = pl.pallas_call(
        _router_kernel,
        out_shape=(
            jax.ShapeDtypeStruct((2 * N, 1), jnp.int32),
            jax.ShapeDtypeStruct((N, 1), jnp.float32),
            jax.ShapeDtypeStruct((N, 1), jnp.float32),
            jax.ShapeDtypeStruct((nh * vmax, 1), jnp.int32),
            jax.ShapeDtypeStruct((nh * vmax, 1), jnp.int32),
            jax.ShapeDtypeStruct((nh * vmax, 1), jnp.int32),
            jax.ShapeDtypeStruct((1, 1), jnp.float32),
        ),
    )(x_flat, latent_code, Wr, Wl, tri)
    return pos, g1, g2, ve, vb, vh, loss[0, 0]


def _sc_dispatch(pos, x_flat, npad):
    N, D = x_flat.shape
    S = 2 * N
    info = plsc.get_sparse_core_info()
    nc, ns = info.num_cores, info.num_subcores
    nw = nc * ns
    per_w = S // nw
    ch = min(per_w, 64)
    mesh = plsc.VectorSubcoreMesh(core_axis_name="c", subcore_axis_name="s")

    @functools.partial(
        pl.kernel,
        out_type=jax.ShapeDtypeStruct((npad, D), jnp.float32),
        mesh=mesh,
        scratch_types=[
            pltpu.VMEM((ch,), jnp.int32),
            pltpu.VMEM((ch, D), jnp.float32),
            pltpu.SemaphoreType.DMA,
        ],
    )
    def scatter(pos_hbm, x_hbm, xs_hbm, idx_v, rows_v, sem):
        wid = lax.axis_index("s") * nc + lax.axis_index("c")
        base = wid * per_w
        for c in range(per_w // ch):
            b = base + c * ch
            srow = lax.rem(b, N)
            pltpu.sync_copy(pos_hbm.at[pl.ds(b, ch)], idx_v)
            pltpu.sync_copy(x_hbm.at[pl.ds(srow, ch)], rows_v)
            pltpu.async_copy(rows_v, xs_hbm.at[idx_v], sem).wait()

    return scatter(pos, x_flat)


def _sc_collect(pos, ys):
    S = pos.shape[0]
    npad, D = ys.shape
    info = plsc.get_sparse_core_info()
    nc, ns = info.num_cores, info.num_subcores
    nw = nc * ns
    per_w = S // nw
    ch = min(per_w, 64)
    mesh = plsc.VectorSubcoreMesh(core_axis_name="c", subcore_axis_name="s")

    @functools.partial(
        pl.kernel,
        out_type=jax.ShapeDtypeStruct((S, D), jnp.float32),
        mesh=mesh,
        scratch_types=[
            pltpu.VMEM((ch,), jnp.int32),
            pltpu.VMEM((ch, D), jnp.float32),
            pltpu.SemaphoreType.DMA,
        ],
    )
    def gather(pos_hbm, ys_hbm, yp_hbm, idx_v, rows_v, sem):
        wid = lax.axis_index("s") * nc + lax.axis_index("c")
        base = wid * per_w
        for c in range(per_w // ch):
            b = base + c * ch
            pltpu.sync_copy(pos_hbm.at[pl.ds(b, ch)], idx_v)
            pltpu.async_copy(ys_hbm.at[idx_v], rows_v, sem).wait()
            pltpu.sync_copy(rows_v, yp_hbm.at[pl.ds(b, ch)])

    return gather(pos, ys)


def _ffn_kernel(ve_ref, vb_ref, vh_ref,
                xs_ref, w1_ref, b1_ref, w2_ref, b2_ref, ys_ref):
    h = pl.program_id(0)
    v = h * (vb_ref.shape[0] // pl.num_programs(0)) + pl.program_id(1)
    block = vb_ref[v]
    hi = vh_ref[v]

    @pl.when(hi > block * BM)
    def _():
        x = xs_ref[...]
        w1 = w1_ref[0]
        hidden = jnp.dot(x, w1.T, preferred_element_type=jnp.float32)
        hidden = _gelu_exact(hidden + b1_ref[0, 0])
        y = jnp.dot(hidden, w2_ref[0].T, preferred_element_type=jnp.float32)

        rows = block * BM + lax.broadcasted_iota(jnp.int32, (BM, 1), 0)
        mask = rows < hi

        @pl.when(h == 0)
        def _():
            ys_ref[pl.ds(block * BM, BM), :] = jnp.where(mask, y + b2_ref[0], 0.0)

        @pl.when(h != 0)
        def _():
            ys_ref[pl.ds(block * BM, BM), :] += jnp.where(mask, y, 0.0)


def _grouped_ffn(xs, ve, vb, vh, W1, b1, W2, b2):
    npad, D = xs.shape
    E, H, _ = W1.shape
    nh = H // BH
    vmax = ve.shape[0] // nh
    b1r = b1.reshape(E, nh, 1, BH)
    b2r = b2.reshape(E, 1, D)
    vm = vmax
    grid_spec = pltpu.PrefetchScalarGridSpec(
        num_scalar_prefetch=3,
        grid=(nh, vmax),
        in_specs=[
            pl.BlockSpec((BM, D), lambda h, v, ve, vb, vh: (vb[h * vm + v], 0)),
            pl.BlockSpec((1, BH, D),
                         lambda h, v, ve, vb, vh: (ve[h * vm + v], h, 0)),
            pl.BlockSpec((1, 1, 1, BH),
                         lambda h, v, ve, vb, vh: (ve[h * vm + v], h, 0, 0)),
            pl.BlockSpec((1, D, BH),
                         lambda h, v, ve, vb, vh: (ve[h * vm + v], 0, h)),
            pl.BlockSpec((1, 1, D),
                         lambda h, v, ve, vb, vh: (ve[h * vm + v], 0, 0)),
        ],
        out_specs=pl.BlockSpec((npad, D), lambda h, v, ve, vb, vh: (0, 0)),
    )
    return pl.pallas_call(
        _ffn_kernel,
        grid_spec=grid_spec,
        out_shape=jax.ShapeDtypeStruct((npad, D), jnp.float32),
    )(ve, vb, vh, xs, W1, b1r, W2, b2r)


def _combine_kernel(y1_ref, y2_ref, g1_ref, g2_ref, out_ref):
    out_ref[...] = g1_ref[...] * y1_ref[...] + g2_ref[...] * y2_ref[...]


def _combine(yp, g1, g2, bt=512):
    S, D = yp.shape
    N = S // 2
    return pl.pallas_call(
        _combine_kernel,
        grid=(N // bt,),
        in_specs=[
            pl.BlockSpec((bt, D), lambda t: (t, 0)),
            pl.BlockSpec((bt, D), lambda t, n=N // bt: (t + n, 0)),
            pl.BlockSpec((bt, 1), lambda t: (t, 0)),
            pl.BlockSpec((bt, 1), lambda t: (t, 0)),
        ],
        out_specs=pl.BlockSpec((bt, D), lambda t: (t, 0)),
        out_shape=jax.ShapeDtypeStruct((N, D), jnp.float32),
    )(yp, yp, g1, g2)


@jax.jit
def kernel(x, latent_code, Wr, Wl, W1, b1, W2, b2):
    B, T, D = x.shape
    E, H, _ = W1.shape
    x_flat = x.reshape(-1, D)
    N = x_flat.shape[0]
    npad = 2 * N + E * BM
    vmax = npad // BM

    nh = H // BH
    pos, g1, g2, ve, vb, vh, loss = _route(x_flat, latent_code, Wr, Wl, vmax, nh)

    pos_flat = pos.reshape(2 * N)
    xs = _sc_dispatch(pos_flat, x_flat, npad)
    ys = _grouped_ffn(xs, ve.reshape(nh * vmax), vb.reshape(nh * vmax),
                      vh.reshape(nh * vmax), W1, b1, W2, b2)
    yp = _sc_collect(pos_flat, ys)
    out = _combine(yp, g1, g2)
    return out.reshape(B, T, D), loss
